# baseline (speedup 1.0000x reference)
import functools

import jax
import jax.numpy as jnp
import numpy as np
from jax import lax
from jax.experimental import pallas as pl
from jax.experimental.pallas import tpu as pltpu
from jax.experimental.pallas import tpu_sc as plsc

_B, _L, _D = 4096, 200, 64
_V = 1000000
_NC, _NS = 2, 16
_FPC = _D // _NC
_NLG, _NBH = 8, 2
_LPT = _L // _NLG
_BPT = _B // _NBH
_EPT = _LPT * _BPT
_LCH = 5
_ECH = _LCH * _BPT
_VMAIN = 999936
_RCH = 62464
_REM = _VMAIN - 16 * _RCH


def _sc_embed_t(table_t, idx_t, posx, tail_t):
    mesh = plsc.VectorSubcoreMesh(core_axis_name="c", subcore_axis_name="s")

    @functools.partial(
        pl.kernel,
        mesh=mesh,
        out_type=jax.ShapeDtypeStruct((_L, _D, _B), jnp.float32),
        scratch_types=[
            pltpu.VMEM((_EPT,), jnp.int32),
            pltpu.VMEM((_ECH,), jnp.float32),
            pltpu.VMEM((32, 128), jnp.float32),
            pltpu.VMEM_SHARED((_VMAIN + 128,), jnp.float32),
            pltpu.SemaphoreType.DMA,
            pltpu.SemaphoreType.DMA,
            pltpu.SemaphoreType.DMA,
        ],
        compiler_params=pltpu.CompilerParams(
            use_tc_tiling_on_sc=True, needs_layout_passes=False),
    )
    def k(table_hbm, idx_hbm, posx_hbm, tail_hbm, out_hbm,
          idx_v, res_v, posx_v, row_sh, gsem, osem, isem):
        c = lax.axis_index("c")
        s = lax.axis_index("s")
        lg = s // _NBH
        bh = s % _NBH
        l0 = lg * _LPT
        b0 = bh * _BPT

        def stage_idx(i, carry):
            pltpu.async_copy(
                idx_hbm.at[l0 + i, pl.ds(b0, _BPT)],
                idx_v.at[pl.ds(i * _BPT, _BPT)], isem)
            return carry

        lax.fori_loop(0, _LPT, stage_idx, 0)
        pltpu.sync_copy(posx_hbm.at[pl.ds(lg * 32, 32)], posx_v)

        def drain_idx(i, carry):
            pltpu.make_async_copy(
                idx_hbm.at[l0, pl.ds(b0, _BPT)],
                idx_v.at[pl.ds(i * _BPT, _BPT)], isem).wait()
            return carry

        lax.fori_loop(0, _LPT, drain_idx, 0)

        def drain_out_one(il):
            pltpu.make_async_copy(
                res_v.at[pl.ds(il * _BPT, _BPT)],
                out_hbm.at[l0, 0, pl.ds(b0, _BPT)], osem).wait()

        def feat_body(fl, carry):
            f = c * _FPC + fl
            plsc.subcore_barrier()
            row_off = s * _RCH
            pltpu.sync_copy(
                table_hbm.at[f, pl.ds(row_off, _RCH)],
                row_sh.at[pl.ds(row_off, _RCH)])

            @pl.when(s == _NS - 1)
            def _():
                pltpu.sync_copy(
                    table_hbm.at[f, pl.ds(16 * _RCH, _REM)],
                    row_sh.at[pl.ds(16 * _RCH, _REM)])
                pltpu.sync_copy(
                    tail_hbm.at[f],
                    row_sh.at[pl.ds(_VMAIN, 128)])

            plsc.subcore_barrier()

            for ch in range(_LPT // _LCH):
                for il in range(_LCH):
                    li = ch * _LCH + il
                    if ch == 0:
                        @pl.when(fl >= 1)
                        def _(il=il):
                            drain_out_one(il)
                    else:
                        drain_out_one(il)
                    for q in range(2):
                        pltpu.async_copy(
                            row_sh.at[idx_v.at[
                                pl.ds(li * _BPT + q * (_BPT // 2),
                                      _BPT // 2)]],
                            res_v.at[pl.ds(il * _BPT + q * (_BPT // 2),
                                           _BPT // 2)], gsem)

                for il in range(_LCH):
                    li = ch * _LCH + il
                    pltpu.make_async_copy(
                        row_sh.at[idx_v.at[pl.ds(li * _BPT, _BPT)]],
                        res_v.at[pl.ds(il * _BPT, _BPT)], gsem).wait()
                    sp = plsc.load_gather(
                        posx_v, [jnp.full((16,), li, jnp.int32),
                                 jnp.full((16,), f, jnp.int32)])

                    def add_j(j, acc, il=il, sp=sp):
                        for k16 in range(16):
                            plsc.addupdate(
                                res_v.at[pl.ds(
                                    il * _BPT + j * 256 + k16 * 16, 16)],
                                sp)
                        return acc

                    lax.fori_loop(0, _BPT // 256, add_j, 0)
                    pltpu.async_copy(
                        res_v.at[pl.ds(il * _BPT, _BPT)],
                        out_hbm.at[l0 + li, f, pl.ds(b0, _BPT)], osem)
            return carry

        lax.fori_loop(0, _FPC, feat_body, 0)
        for il in range(_LCH):
            drain_out_one(il)

    return k(table_t, idx_t, posx, tail_t)


_SLOT_L = np.minimum((np.arange(256) // 32) * _LPT
                     + np.minimum(np.arange(256) % 32, _LPT - 1), _L - 1)


def kernel(input, emb_table, pos_table):
    tail_t = jnp.pad(emb_table[_VMAIN:], ((0, 128 - (_V - _VMAIN)), (0, 0))).T
    posx = jnp.pad(pos_table[:_L], ((0, 0), (0, 128 - _D)))[_SLOT_L]
    out_t = _sc_embed_t(emb_table.T, input.T, posx, tail_t)
    return out_t.transpose(2, 0, 1)


# speedup vs baseline: 1.0005x; 1.0005x over previous
import functools

import jax
import jax.numpy as jnp
import numpy as np
from jax import lax
from jax.experimental import pallas as pl
from jax.experimental.pallas import tpu as pltpu
from jax.experimental.pallas import tpu_sc as plsc

_B, _L, _D = 4096, 200, 64
_V = 1000000
_NC, _NS = 2, 16
_FPC = _D // _NC
_NLG, _NBH = 8, 2
_LPT = _L // _NLG
_BPT = _B // _NBH
_EPT = _LPT * _BPT
_LCH = 5
_ECH = _LCH * _BPT
_VMAIN = 999936
_RCH = 62464
_REM = _VMAIN - 16 * _RCH


def _sc_embed_t(table_t, idx_t, posx, tail_t):
    mesh = plsc.VectorSubcoreMesh(core_axis_name="c", subcore_axis_name="s")

    @functools.partial(
        pl.kernel,
        mesh=mesh,
        out_type=jax.ShapeDtypeStruct((_L, _D, _B), jnp.float32),
        scratch_types=[
            pltpu.VMEM((_EPT,), jnp.int32),
            pltpu.VMEM((_ECH,), jnp.float32),
            pltpu.VMEM((32, 128), jnp.float32),
            pltpu.VMEM_SHARED((_VMAIN + 128,), jnp.float32),
            pltpu.SemaphoreType.DMA,
            pltpu.SemaphoreType.DMA,
            pltpu.SemaphoreType.DMA,
        ],
        compiler_params=pltpu.CompilerParams(
            use_tc_tiling_on_sc=True, needs_layout_passes=False),
    )
    def k(table_hbm, idx_hbm, posx_hbm, tail_hbm, out_hbm,
          idx_v, res_v, posx_v, row_sh, gsem, osem, isem):
        c = lax.axis_index("c")
        s = lax.axis_index("s")
        lg = s // _NBH
        bh = s % _NBH
        l0 = lg * _LPT
        b0 = bh * _BPT

        def stage_idx(i, carry):
            pltpu.async_copy(
                idx_hbm.at[l0 + i, pl.ds(b0, _BPT)],
                idx_v.at[pl.ds(i * _BPT, _BPT)], isem)
            return carry

        lax.fori_loop(0, _LPT, stage_idx, 0)
        pltpu.sync_copy(posx_hbm.at[pl.ds(lg * 32, 32)], posx_v)

        def drain_idx(i, carry):
            pltpu.make_async_copy(
                idx_hbm.at[l0, pl.ds(b0, _BPT)],
                idx_v.at[pl.ds(i * _BPT, _BPT)], isem).wait()
            return carry

        lax.fori_loop(0, _LPT, drain_idx, 0)

        def drain_out_one(il):
            pltpu.make_async_copy(
                res_v.at[pl.ds(il * _BPT, _BPT)],
                out_hbm.at[l0, 0, pl.ds(b0, _BPT)], osem).wait()

        def feat_body(fl, carry):
            f = c * _FPC + fl
            plsc.subcore_barrier()
            row_off = s * _RCH
            cps = [
                pltpu.async_copy(
                    table_hbm.at[f, pl.ds(row_off + q * (_RCH // 2),
                                          _RCH // 2)],
                    row_sh.at[pl.ds(row_off + q * (_RCH // 2), _RCH // 2)],
                    isem)
                for q in range(2)
            ]
            for cp in cps:
                cp.wait()

            @pl.when(s == _NS - 1)
            def _():
                pltpu.sync_copy(
                    table_hbm.at[f, pl.ds(16 * _RCH, _REM)],
                    row_sh.at[pl.ds(16 * _RCH, _REM)])
                pltpu.sync_copy(
                    tail_hbm.at[f],
                    row_sh.at[pl.ds(_VMAIN, 128)])

            plsc.subcore_barrier()

            for ch in range(_LPT // _LCH):
                for il in range(_LCH):
                    li = ch * _LCH + il
                    if ch == 0:
                        @pl.when(fl >= 1)
                        def _(il=il):
                            drain_out_one(il)
                    else:
                        drain_out_one(il)
                    pltpu.async_copy(
                        row_sh.at[idx_v.at[pl.ds(li * _BPT, _BPT)]],
                        res_v.at[pl.ds(il * _BPT, _BPT)], gsem)

                for il in range(_LCH):
                    li = ch * _LCH + il
                    pltpu.make_async_copy(
                        row_sh.at[idx_v.at[pl.ds(li * _BPT, _BPT)]],
                        res_v.at[pl.ds(il * _BPT, _BPT)], gsem).wait()
                    sp = plsc.load_gather(
                        posx_v, [jnp.full((16,), li, jnp.int32),
                                 jnp.full((16,), f, jnp.int32)])

                    def add_j(j, acc, il=il, sp=sp):
                        for k16 in range(16):
                            plsc.addupdate(
                                res_v.at[pl.ds(
                                    il * _BPT + j * 256 + k16 * 16, 16)],
                                sp)
                        return acc

                    lax.fori_loop(0, _BPT // 256, add_j, 0)
                    pltpu.async_copy(
                        res_v.at[pl.ds(il * _BPT, _BPT)],
                        out_hbm.at[l0 + li, f, pl.ds(b0, _BPT)], osem)
            return carry

        lax.fori_loop(0, _FPC, feat_body, 0)
        for il in range(_LCH):
            drain_out_one(il)

    return k(table_t, idx_t, posx, tail_t)


_SLOT_L = np.minimum((np.arange(256) // 32) * _LPT
                     + np.minimum(np.arange(256) % 32, _LPT - 1), _L - 1)


def kernel(input, emb_table, pos_table):
    tail_t = jnp.pad(emb_table[_VMAIN:], ((0, 128 - (_V - _VMAIN)), (0, 0))).T
    posx = jnp.pad(pos_table[:_L], ((0, 0), (0, 128 - _D)))[_SLOT_L]
    out_t = _sc_embed_t(emb_table.T, input.T, posx, tail_t)
    return out_t.transpose(2, 0, 1)


# speedup vs baseline: 1.0039x; 1.0034x over previous
import functools

import jax
import jax.numpy as jnp
import numpy as np
from jax import lax
from jax.experimental import pallas as pl
from jax.experimental.pallas import tpu as pltpu
from jax.experimental.pallas import tpu_sc as plsc

_B, _L, _D = 4096, 200, 64
_V = 1000000
_NC, _NS = 2, 16
_FPC = _D // _NC
_NLG, _NBH = 8, 2
_LPT = _L // _NLG
_BPT = _B // _NBH
_EPT = _LPT * _BPT
_LCH = 5
_ECH = _LCH * _BPT
_VMAIN = 999936
_RCH = 62464
_REM = _VMAIN - 16 * _RCH


def _sc_embed_t(table_t, idx_t, posx, tail_t):
    mesh = plsc.VectorSubcoreMesh(core_axis_name="c", subcore_axis_name="s")

    @functools.partial(
        pl.kernel,
        mesh=mesh,
        out_type=jax.ShapeDtypeStruct((_L, _D, _B), jnp.float32),
        scratch_types=[
            pltpu.VMEM((_EPT,), jnp.int32),
            pltpu.VMEM((_ECH,), jnp.float32),
            pltpu.VMEM((32, 128), jnp.float32),
            pltpu.VMEM_SHARED((_VMAIN + 128,), jnp.float32),
            pltpu.SemaphoreType.DMA,
            pltpu.SemaphoreType.DMA,
            pltpu.SemaphoreType.DMA,
        ],
        compiler_params=pltpu.CompilerParams(
            use_tc_tiling_on_sc=True, needs_layout_passes=False),
    )
    def k(table_hbm, idx_hbm, posx_hbm, tail_hbm, out_hbm,
          idx_v, res_v, posx_v, row_sh, gsem, osem, isem):
        c = lax.axis_index("c")
        s = lax.axis_index("s")
        lg = s // _NBH
        bh = s % _NBH
        l0 = lg * _LPT
        b0 = bh * _BPT

        def stage_idx(i, carry):
            pltpu.async_copy(
                idx_hbm.at[l0 + i, pl.ds(b0, _BPT)],
                idx_v.at[pl.ds(i * _BPT, _BPT)], isem)
            return carry

        lax.fori_loop(0, _LPT, stage_idx, 0)
        pltpu.sync_copy(posx_hbm.at[pl.ds(lg * 32, 32)], posx_v)

        def drain_idx(i, carry):
            pltpu.make_async_copy(
                idx_hbm.at[l0, pl.ds(b0, _BPT)],
                idx_v.at[pl.ds(i * _BPT, _BPT)], isem).wait()
            return carry

        lax.fori_loop(0, _LPT, drain_idx, 0)

        def drain_out_one(il):
            pltpu.make_async_copy(
                res_v.at[pl.ds(il * _BPT, _BPT)],
                out_hbm.at[l0, 0, pl.ds(b0, _BPT)], osem).wait()

        def feat_body(fl, carry):
            f = c * _FPC + fl
            plsc.subcore_barrier()
            row_off = s * _RCH
            pltpu.sync_copy(
                table_hbm.at[f, pl.ds(row_off, _RCH)],
                row_sh.at[pl.ds(row_off, _RCH)])

            @pl.when(s == _NS - 1)
            def _():
                pltpu.sync_copy(
                    table_hbm.at[f, pl.ds(16 * _RCH, _REM)],
                    row_sh.at[pl.ds(16 * _RCH, _REM)])
                pltpu.sync_copy(
                    tail_hbm.at[f],
                    row_sh.at[pl.ds(_VMAIN, 128)])

            plsc.subcore_barrier()

            for ch in range(_LPT // _LCH):
                for il in range(_LCH):
                    li = ch * _LCH + il
                    if ch == 0:
                        @pl.when(fl >= 1)
                        def _(il=il):
                            drain_out_one(il)
                    else:
                        drain_out_one(il)
                    pltpu.async_copy(
                        row_sh.at[idx_v.at[pl.ds(li * _BPT, _BPT)]],
                        res_v.at[pl.ds(il * _BPT, _BPT)], gsem)

                for il in range(_LCH):
                    li = ch * _LCH + il
                    pltpu.make_async_copy(
                        row_sh.at[idx_v.at[pl.ds(li * _BPT, _BPT)]],
                        res_v.at[pl.ds(il * _BPT, _BPT)], gsem).wait()
                    sp = plsc.load_gather(
                        posx_v, [jnp.full((16,), li, jnp.int32),
                                 jnp.full((16,), f, jnp.int32)])

                    def add_j(j, acc, il=il, sp=sp):
                        for k16 in range(16):
                            plsc.addupdate(
                                res_v.at[pl.ds(
                                    il * _BPT + j * 256 + k16 * 16, 16)],
                                sp)
                        return acc

                    lax.fori_loop(0, _BPT // 256, add_j, 0)
                    pltpu.async_copy(
                        res_v.at[pl.ds(il * _BPT, _BPT)],
                        out_hbm.at[l0 + li, f, pl.ds(b0, _BPT)], osem)
            return carry

        lax.fori_loop(0, _FPC, feat_body, 0)
        for il in range(_LCH):
            drain_out_one(il)

    return k(table_t, idx_t, posx, tail_t)


_SLOT_L = np.minimum((np.arange(256) // 32) * _LPT
                     + np.minimum(np.arange(256) % 32, _LPT - 1), _L - 1)


def kernel(input, emb_table, pos_table):
    tail_t = jnp.pad(emb_table[_VMAIN:], ((0, 128 - (_V - _VMAIN)), (0, 0))).T
    posx = jnp.pad(pos_table[:_L], ((0, 0), (0, 128 - _D)))[_SLOT_L]
    out_t = _sc_embed_t(emb_table.T, input.T, posx, tail_t)
    return out_t.transpose(2, 0, 1)
